# baseline (speedup 1.0000x reference)
import functools
import math

import jax
import jax.numpy as jnp
from jax import lax
from jax.experimental import pallas as pl
from jax.experimental.pallas import tpu as pltpu
from jax.experimental.pallas import tpu_sc as plsc

_VOCAB = 1000000
_EMB = 64
_B = 4096
_L = 200
_SCALE = math.sqrt(_EMB)

_NC = 2
_NS = 16
_NW = _NC * _NS

_N = _B * _L
_PER_W = _N // _NW
_CHUNK = 128
_NCHUNK = _PER_W // _CHUNK
_NBUF = 4
_NOUTER = _NCHUNK // _NBUF


def _emb_body(tokens_hbm, table_hbm, out_hbm, idx_v, rows_in, rows_out, *sems):
    sem_g = sems[:_NBUF]
    sem_o = sems[_NBUF:]
    wid = lax.axis_index("s") * _NC + lax.axis_index("c")
    base = wid * _PER_W

    pltpu.sync_copy(tokens_hbm.at[wid], idx_v)

    for b in range(_NBUF):
        pltpu.make_async_copy(
            table_hbm.at[idx_v.at[b]], rows_in.at[b], sem_g[b]
        ).start()

    def round_body(g, carry):
        for b in range(_NBUF):
            c = g * _NBUF + b
            pltpu.make_async_copy(
                table_hbm.at[idx_v.at[c]], rows_in.at[b], sem_g[b]
            ).wait()

            @pl.when(g > 0)
            def _wait_out():
                pltpu.make_async_copy(
                    rows_out.at[b], out_hbm.at[pl.ds(base, _CHUNK)], sem_o[b]
                ).wait()

            def scale_row(r, acc):
                for j in range(4):
                    rows_out[b, r, pl.ds(j * 16, 16)] = (
                        rows_in[b, r, pl.ds(j * 16, 16)] * _SCALE
                    )
                return acc

            lax.fori_loop(0, _CHUNK, scale_row, 0, unroll=8)

            pltpu.make_async_copy(
                rows_out.at[b], out_hbm.at[pl.ds(base + c * _CHUNK, _CHUNK)], sem_o[b]
            ).start()

            @pl.when(g < _NOUTER - 1)
            def _next_gather():
                pltpu.make_async_copy(
                    table_hbm.at[idx_v.at[c + _NBUF]], rows_in.at[b], sem_g[b]
                ).start()

        return carry

    lax.fori_loop(0, _NOUTER, round_body, 0)

    for b in range(_NBUF):
        pltpu.make_async_copy(
            rows_out.at[b], out_hbm.at[pl.ds(base, _CHUNK)], sem_o[b]
        ).wait()


@jax.jit
def _embed(tokens32, table):
    mesh = plsc.VectorSubcoreMesh(core_axis_name="c", subcore_axis_name="s")
    run = pl.kernel(
        _emb_body,
        out_type=jax.ShapeDtypeStruct((_N, _EMB), jnp.float32),
        mesh=mesh,
        scratch_types=(
            [
                pltpu.VMEM((_NCHUNK, _CHUNK), jnp.int32),
                pltpu.VMEM((_NBUF, _CHUNK, _EMB), jnp.float32),
                pltpu.VMEM((_NBUF, _CHUNK, _EMB), jnp.float32),
            ]
            + [pltpu.SemaphoreType.DMA] * (2 * _NBUF)
        ),
        compiler_params=pltpu.CompilerParams(use_tc_tiling_on_sc=False),
    )
    return run(tokens32, table)


def kernel(tokens, table):
    tok = tokens.astype(jnp.int32).reshape(_NW, _NCHUNK, _CHUNK)
    out = _embed(tok, table)
    return out.reshape(_B, _L, _EMB)
